# baseline (speedup 1.0000x reference)
import functools

import jax
import jax.numpy as jnp
from jax import lax
from jax.experimental import pallas as pl
from jax.experimental.pallas import tpu as pltpu
from jax.experimental.pallas import tpu_sc as plsc

VOCAB = 1000000
DIM = 32
BATCH = 16384
_ROWS_PER_TILE = 8
_NTILES = VOCAB // _ROWS_PER_TILE

_info = plsc.get_sparse_core_info()
_NC, _NS, _L = _info.num_cores, _info.num_subcores, _info.num_lanes
_NW = _NC * _NS
_BPW = BATCH // _NW


def _make_gather():
    mesh = plsc.VectorSubcoreMesh(core_axis_name="c", subcore_axis_name="s")

    @functools.partial(
        pl.kernel,
        mesh=mesh,
        out_type=jax.ShapeDtypeStruct((BATCH, DIM), jnp.float32),
        scratch_types=[
            pltpu.VMEM((_BPW,), jnp.int32),
            pltpu.VMEM((_BPW, DIM), jnp.float32),
            pltpu.SemaphoreType.DMA,
        ],
    )
    def gather(table_hbm, idx_hbm, out_hbm, idx_v, rows_v, sem):
        wid = lax.axis_index("s") * _NC + lax.axis_index("c")
        base = wid * _BPW
        pltpu.sync_copy(idx_hbm.at[pl.ds(base, _BPW)], idx_v)

        def body(g, carry):
            iv = idx_v[pl.ds(g * _L, _L)]
            for l in range(_L):
                ix = iv[l]
                pltpu.async_copy(table_hbm.at[ix], rows_v.at[g * _L + l], sem)
            return carry

        lax.fori_loop(0, _BPW // _L, body, 0)
        pltpu.make_async_copy(out_hbm.at[pl.ds(base, _BPW)], rows_v, sem).wait()
        pltpu.sync_copy(rows_v, out_hbm.at[pl.ds(base, _BPW)])

    return gather


_gather = _make_gather()


@jax.jit
def kernel(warehouse_id, table):
    return _gather(table, warehouse_id)


# speedup vs baseline: 1.6096x; 1.6096x over previous
import functools

import jax
import jax.numpy as jnp
from jax import lax
from jax.experimental import pallas as pl
from jax.experimental.pallas import tpu as pltpu
from jax.experimental.pallas import tpu_sc as plsc

VOCAB = 1000000
DIM = 32
BATCH = 16384
_ROWS_PER_TILE = 8
_NTILES = VOCAB // _ROWS_PER_TILE

_info = plsc.get_sparse_core_info()
_NC, _NS, _L = _info.num_cores, _info.num_subcores, _info.num_lanes
_NW = _NC * _NS
_BPW = BATCH // _NW


def _make_gather():
    mesh = plsc.VectorSubcoreMesh(core_axis_name="c", subcore_axis_name="s")

    @functools.partial(
        pl.kernel,
        mesh=mesh,
        out_type=jax.ShapeDtypeStruct((DIM, BATCH), jnp.float32),
        scratch_types=[
            pltpu.VMEM((_BPW,), jnp.int32),
            pltpu.VMEM((_BPW, DIM), jnp.float32),
            pltpu.VMEM((DIM, _BPW), jnp.float32),
            pltpu.SemaphoreType.DMA,
        ],
        compiler_params=pltpu.CompilerParams(needs_layout_passes=False),
    )
    def gather(table3_hbm, idx_hbm, out_hbm, idx_v, rows_v, tbuf_v, sem):
        wid = lax.axis_index("s") * _NC + lax.axis_index("c")
        base = wid * _BPW
        pltpu.sync_copy(idx_hbm.at[pl.ds(base, _BPW)], idx_v)

        def body(g, carry):
            iv = idx_v[pl.ds(g * _L, _L)]
            for l in range(_L):
                ix = iv[l]
                t = lax.shift_right_logical(ix, 3)
                r = lax.bitwise_and(ix, 7)
                pltpu.async_copy(table3_hbm.at[t, r], rows_v.at[g * _L + l], sem)
            return carry

        lax.fori_loop(0, _BPW // _L, body, 0)
        def dbody(g, carry):
            pltpu.make_async_copy(table3_hbm.at[0, 0], rows_v.at[0], sem).wait()
            return carry

        lax.fori_loop(0, _BPW, dbody, 0)

        def tbody(g, carry):
            j_vec = lax.iota(jnp.int32, _L) + g * _L
            for c in range(DIM):
                c_vec = jnp.full((_L,), c, dtype=jnp.int32)
                vals = plsc.load_gather(rows_v, [j_vec, c_vec])
                plsc.store_scatter(tbuf_v, [c_vec, j_vec], vals)
            return carry

        lax.fori_loop(0, _BPW // _L, tbody, 0)
        pltpu.sync_copy(tbuf_v, out_hbm.at[:, pl.ds(base, _BPW)])

    return gather


_gather = _make_gather()


@jax.jit
def kernel(warehouse_id, table):
    table3 = table.reshape(_NTILES, _ROWS_PER_TILE, DIM)
    out_t = _gather(table3, warehouse_id)
    return out_t.T


# speedup vs baseline: 1.6570x; 1.0294x over previous
import functools

import jax
import jax.numpy as jnp
from jax import lax
from jax.experimental import pallas as pl
from jax.experimental.pallas import tpu as pltpu
from jax.experimental.pallas import tpu_sc as plsc

VOCAB = 1000000
DIM = 32
BATCH = 16384
_ROWS_PER_TILE = 8
_NTILES = VOCAB // _ROWS_PER_TILE

_info = plsc.get_sparse_core_info()
_NC, _NS, _L = _info.num_cores, _info.num_subcores, _info.num_lanes
_NW = _NC * _NS
_BPW = BATCH // _NW


def _make_gather():
    mesh = plsc.VectorSubcoreMesh(core_axis_name="c", subcore_axis_name="s")

    @functools.partial(
        pl.kernel,
        mesh=mesh,
        out_type=jax.ShapeDtypeStruct((BATCH, DIM), jnp.float32),
        scratch_types=[
            pltpu.VMEM((_BPW,), jnp.int32),
            pltpu.VMEM((_BPW, DIM), jnp.float32),
            pltpu.SemaphoreType.DMA,
        ],
        compiler_params=pltpu.CompilerParams(needs_layout_passes=False),
    )
    def gather(table3_hbm, idx_hbm, out_hbm, idx_v, rows_v, sem):
        wid = lax.axis_index("s") * _NC + lax.axis_index("c")
        base = wid * _BPW
        pltpu.sync_copy(idx_hbm.at[pl.ds(base, _BPW)], idx_v)

        def body(g, carry):
            iv = idx_v[pl.ds(g * _L, _L)]
            for l in range(_L):
                ix = iv[l]
                t = lax.shift_right_logical(ix, 3)
                r = lax.bitwise_and(ix, 7)
                pltpu.async_copy(table3_hbm.at[t, r], rows_v.at[g * _L + l], sem)
            return carry

        lax.fori_loop(0, _BPW // _L, body, 0)
        pltpu.make_async_copy(out_hbm.at[pl.ds(base, _BPW)], rows_v, sem).wait()
        pltpu.sync_copy(rows_v, out_hbm.at[pl.ds(base, _BPW)])

    return gather


_gather = _make_gather()


@jax.jit
def kernel(warehouse_id, table):
    table3 = table.reshape(_NTILES, _ROWS_PER_TILE, DIM)
    return _gather(table3, warehouse_id)
